# baseline (speedup 1.0000x reference)
import jax
import jax.numpy as jnp
from jax.experimental import pallas as pl

HID = 64
NTYPES = 3
ITERS = 2
BLK = 2000
SROW = 80


def _gate_tab_body(pt_ref, wp_ref, bp_ref, out_ref):
    z = jnp.dot(pt_ref[...], wp_ref[...], preferred_element_type=jnp.float32)
    out_ref[...] = 2.0 * jax.nn.sigmoid(z + bp_ref[...])


def _node_body(s_ref, h_ref, we_ref, be_ref, wih_ref, whh_ref, bih_ref,
               bhh_ref, out_ref):
    h = h_ref[...]
    sums = jnp.zeros((BLK, HID), jnp.float32)
    cnt = jnp.zeros((BLK, 1), jnp.float32)
    for t in range(NTYPES):
        feat = s_ref[t, :, :HID]
        c = s_ref[t, :, HID:HID + 1]
        sums = sums + jnp.dot(feat, we_ref[t],
                              preferred_element_type=jnp.float32)
        sums = sums + c * be_ref[t][None, :]
        cnt = cnt + c
    agg = sums / jnp.maximum(cnt, 1.0)
    gi = jnp.dot(agg, wih_ref[...], preferred_element_type=jnp.float32) \
        + bih_ref[...]
    gh = jnp.dot(h, whh_ref[...], preferred_element_type=jnp.float32) \
        + bhh_ref[...]
    r = jax.nn.sigmoid(gi[:, :HID] + gh[:, :HID])
    z = jax.nn.sigmoid(gi[:, HID:2 * HID] + gh[:, HID:2 * HID])
    n = jnp.tanh(gi[:, 2 * HID:] + r * gh[:, 2 * HID:])
    out_ref[...] = (1.0 - z) * n + z * h


def _mlp_body(h_ref, h0_ref, w1_ref, b1_ref, w2_ref, b2_ref, out_ref):
    comb = jnp.concatenate([h_ref[...], h0_ref[...]], axis=-1)
    hid = jax.nn.relu(jnp.dot(comb, w1_ref[...],
                              preferred_element_type=jnp.float32) + b1_ref[...])
    out_ref[...] = jnp.dot(hid, w2_ref[...],
                           preferred_element_type=jnp.float32) + b2_ref[...]


def kernel(x_text_indices, node_selectors, edge_index, edge_type,
           edge_positions, emb, pos_table, Wp, bp, We, be, W_ih, W_hh,
           b_ih, b_hh, W1, b1, W2, b2):
    n_nodes = x_text_indices.shape[0]
    src = edge_index[0]
    dst = edge_index[1]
    nb = n_nodes // BLK

    ptab = jnp.pad(pos_table, ((0, 80 - pos_table.shape[0]), (0, 0)))
    gate_tab = pl.pallas_call(
        _gate_tab_body,
        out_shape=jax.ShapeDtypeStruct((80, HID), jnp.float32),
    )(ptab, Wp, bp.reshape(1, HID))

    h0 = jnp.concatenate([jnp.take(emb, x_text_indices, axis=0),
                          node_selectors], axis=1)

    gate = jnp.take(gate_tab, edge_positions, axis=0)

    def edge_phase(h):
        gx = jnp.take(h, src, axis=0) * gate
        parts = []
        for t in range(NTYPES):
            m = edge_type == t
            feat = jax.ops.segment_sum(jnp.where(m[:, None], gx, 0.0), dst,
                                       num_segments=n_nodes)
            c = jax.ops.segment_sum(m.astype(jnp.float32), dst,
                                    num_segments=n_nodes)
            parts.append(jnp.concatenate(
                [feat, c[:, None], jnp.zeros((n_nodes, SROW - HID - 1),
                                             jnp.float32)], axis=1))
        return jnp.stack(parts)

    node_fn = pl.pallas_call(
        _node_body,
        grid=(nb,),
        in_specs=[
            pl.BlockSpec((NTYPES, BLK, SROW), lambda i: (0, i, 0)),
            pl.BlockSpec((BLK, HID), lambda i: (i, 0)),
            pl.BlockSpec((NTYPES, HID, HID), lambda i: (0, 0, 0)),
            pl.BlockSpec((NTYPES, HID), lambda i: (0, 0)),
            pl.BlockSpec((HID, 3 * HID), lambda i: (0, 0)),
            pl.BlockSpec((HID, 3 * HID), lambda i: (0, 0)),
            pl.BlockSpec((1, 3 * HID), lambda i: (0, 0)),
            pl.BlockSpec((1, 3 * HID), lambda i: (0, 0)),
        ],
        out_specs=pl.BlockSpec((BLK, HID), lambda i: (i, 0)),
        out_shape=jax.ShapeDtypeStruct((n_nodes, HID), jnp.float32),
    )

    h = h0
    wih_t = W_ih.T
    whh_t = W_hh.T
    for _ in range(ITERS):
        s = edge_phase(h)
        h = node_fn(s, h, We, be, wih_t, whh_t,
                    b_ih.reshape(1, -1), b_hh.reshape(1, -1))

    logits = pl.pallas_call(
        _mlp_body,
        grid=(nb,),
        in_specs=[
            pl.BlockSpec((BLK, HID), lambda i: (i, 0)),
            pl.BlockSpec((BLK, HID), lambda i: (i, 0)),
            pl.BlockSpec((2 * HID, HID), lambda i: (0, 0)),
            pl.BlockSpec((1, HID), lambda i: (0, 0)),
            pl.BlockSpec((HID, 1), lambda i: (0, 0)),
            pl.BlockSpec((1, 1), lambda i: (0, 0)),
        ],
        out_specs=pl.BlockSpec((BLK, 1), lambda i: (i, 0)),
        out_shape=jax.ShapeDtypeStruct((n_nodes, 1), jnp.float32),
    )(h, h0, W1, b1.reshape(1, -1), W2, b2.reshape(1, 1))

    return logits


# speedup vs baseline: 1.5195x; 1.5195x over previous
import functools
import jax
import jax.numpy as jnp
from jax import lax
from jax.experimental import pallas as pl
from jax.experimental.pallas import tpu as pltpu, tpu_sc as plsc

HID = 64
NTYPES = 3
ITERS = 2
BLK = 2000
SROW = 80

N_NODES = 50000
N_EDGES = 800000
NW = 32
GBLK = 1600
NBLK = N_EDGES // GBLK
NPASS = 4
RANGE = 6256
ACC_W = RANGE * NTYPES
ACC_ROWS = ACC_W + 16
TRASH = ACC_W
ZROWS = 1176
ZTAIL = ACC_ROWS - 15 * ZROWS
WROWS = 1176
WTAIL = ACC_W - 15 * WROWS
S_ROWS = ACC_W * NPASS * 2
CBUF = GBLK + 192


def _edge_body(idx3_hbm, src_hbm, pos_hbm, h_hbm, gtab_hbm, zeros_hbm,
               s_hbm, idx3_v, src_v, pos_v, cidx, csrc, cpos,
               csrc_b, cidx_b, xbuf, gtab_v, acc, sem):
    c = lax.axis_index("c")
    sid = lax.axis_index("s")

    pltpu.sync_copy(gtab_hbm, gtab_v)

    def per_group(g, off):
        pv = cpos[pl.ds(off + g * 16, 16)]
        for ei in range(16):
            e = g * 16 + ei
            p = pv[ei]
            for q in range(4):
                sl = pl.ds(q * 16, 16)
                xbuf[e, sl] = xbuf[e, sl] * gtab_v[p, sl]
        return off

    def sub_block(sb, _):
        off = sb * 128

        def stage(g, _g):
            sl16 = pl.ds(g * 16, 16)
            csrc_b[sl16] = csrc[pl.ds(off + g * 16, 16)]
            cidx_b[sl16] = cidx[pl.ds(off + g * 16, 16)]
            return 0

        lax.fori_loop(0, 8, stage, 0)
        pltpu.async_copy(h_hbm.at[csrc_b], xbuf, sem).wait()
        lax.fori_loop(0, 8, per_group, off)
        pltpu.sync_copy(xbuf, acc.at[cidx_b], add=True)
        return 0

    def run_pass(p, _):
        r = c * NPASS + p
        lo = r * ACC_W
        hi = lo + ACC_W

        @pl.when(sid < 15)
        def _():
            pltpu.sync_copy(zeros_hbm, acc.at[pl.ds(sid * ZROWS, ZROWS)])

        @pl.when(sid == 15)
        def _():
            pltpu.sync_copy(zeros_hbm.at[pl.ds(0, ZTAIL)],
                            acc.at[pl.ds(15 * ZROWS, ZTAIL)])

        plsc.subcore_barrier()

        def block(k, _):
            b = sid + 16 * k
            base = b * GBLK
            pltpu.sync_copy(idx3_hbm.at[pl.ds(base, GBLK)], idx3_v)
            pltpu.sync_copy(src_hbm.at[pl.ds(base, GBLK)], src_v)
            pltpu.sync_copy(pos_hbm.at[pl.ds(base, GBLK)], pos_v)

            def compact(j, cur):
                sl = pl.ds(j * 16, 16)
                iv = idx3_v[sl]
                m = (iv >= lo) & (iv < hi)
                pfx = plsc.cumsum(m.astype(jnp.int32))
                tgt = cur + pfx - 1
                plsc.store_scatter(cidx, [tgt], iv - lo, mask=m)
                plsc.store_scatter(csrc, [tgt], src_v[sl], mask=m)
                plsc.store_scatter(cpos, [tgt], pos_v[sl], mask=m)
                return cur + pfx[15]

            kcnt = lax.fori_loop(0, GBLK // 16, compact, 0)
            padk = ((kcnt + 127) // 128) * 128

            def pad_tail(j, _):
                off = kcnt + j * 16
                tgt = off + lax.iota(jnp.int32, 16)
                plsc.store_scatter(cidx, [tgt],
                                   jnp.full((16,), TRASH, jnp.int32))
                plsc.store_scatter(csrc, [tgt],
                                   jnp.zeros((16,), jnp.int32))
                plsc.store_scatter(cpos, [tgt],
                                   jnp.zeros((16,), jnp.int32))
                return 0

            lax.fori_loop(0, (padk - kcnt + 15) // 16, pad_tail, 0)
            lax.fori_loop(0, padk // 128, sub_block, 0)
            return 0

        nk = 31 + jnp.where(sid < NBLK - 31 * 16, 1, 0)
        lax.fori_loop(0, nk, block, 0)
        plsc.subcore_barrier()

        @pl.when(sid < 15)
        def _():
            pltpu.sync_copy(acc.at[pl.ds(sid * WROWS, WROWS)],
                            s_hbm.at[pl.ds(lo + sid * WROWS, WROWS)])

        @pl.when(sid == 15)
        def _():
            pltpu.sync_copy(acc.at[pl.ds(15 * WROWS, WTAIL)],
                            s_hbm.at[pl.ds(lo + 15 * WROWS, WTAIL)])

        plsc.subcore_barrier()
        return 0

    lax.fori_loop(0, NPASS, run_pass, 0)


def _make_edge_fn():
    mesh = plsc.VectorSubcoreMesh(core_axis_name="c", subcore_axis_name="s")
    return pl.kernel(
        _edge_body,
        out_type=jax.ShapeDtypeStruct((S_ROWS, SROW), jnp.float32),
        mesh=mesh,
        compiler_params=pltpu.CompilerParams(use_tc_tiling_on_sc=False,
                                             needs_layout_passes=False),
        scratch_types=[
            pltpu.VMEM((GBLK,), jnp.int32),
            pltpu.VMEM((GBLK,), jnp.int32),
            pltpu.VMEM((GBLK,), jnp.int32),
            pltpu.VMEM((CBUF,), jnp.int32),
            pltpu.VMEM((CBUF,), jnp.int32),
            pltpu.VMEM((CBUF,), jnp.int32),
            pltpu.VMEM((128,), jnp.int32),
            pltpu.VMEM((128,), jnp.int32),
            pltpu.VMEM((128, SROW), jnp.float32),
            pltpu.VMEM((SROW, SROW), jnp.float32),
            pltpu.VMEM_SHARED((ACC_ROWS, SROW), jnp.float32),
            pltpu.SemaphoreType.DMA,
        ],
    )


def _gate_tab_body(pt_ref, wp_ref, bp_ref, out_ref):
    z = jnp.dot(pt_ref[...], wp_ref[...], preferred_element_type=jnp.float32)
    g = 2.0 * jax.nn.sigmoid(z + bp_ref[...])
    out_ref[...] = jnp.concatenate(
        [g, jnp.ones((SROW, 1), jnp.float32),
         jnp.zeros((SROW, SROW - HID - 1), jnp.float32)], axis=1)


def _node_body(s_ref, h_ref, we_ref, be_ref, wih_ref, whh_ref, bih_ref,
               bhh_ref, out_ref):
    h = h_ref[:, :HID]
    sums = jnp.zeros((BLK, HID), jnp.float32)
    cnt = jnp.zeros((BLK, 1), jnp.float32)
    for t in range(NTYPES):
        feat = s_ref[:, t, :HID]
        ct = s_ref[:, t, HID:HID + 1]
        sums = sums + jnp.dot(feat, we_ref[t],
                              preferred_element_type=jnp.float32)
        sums = sums + ct * be_ref[t][None, :]
        cnt = cnt + ct
    agg = sums / jnp.maximum(cnt, 1.0)
    gi = jnp.dot(agg, wih_ref[...], preferred_element_type=jnp.float32) \
        + bih_ref[...]
    gh = jnp.dot(h, whh_ref[...], preferred_element_type=jnp.float32) \
        + bhh_ref[...]
    r = jax.nn.sigmoid(gi[:, :HID] + gh[:, :HID])
    z = jax.nn.sigmoid(gi[:, HID:2 * HID] + gh[:, HID:2 * HID])
    n = jnp.tanh(gi[:, 2 * HID:] + r * gh[:, 2 * HID:])
    hn = (1.0 - z) * n + z * h
    out_ref[...] = jnp.concatenate(
        [hn, jnp.ones((BLK, 1), jnp.float32),
         jnp.zeros((BLK, SROW - HID - 1), jnp.float32)], axis=1)


def _mlp_body(h_ref, h0_ref, w1_ref, b1_ref, w2_ref, b2_ref, out_ref):
    comb = jnp.concatenate([h_ref[:, :HID], h0_ref[:, :HID]], axis=-1)
    hid = jax.nn.relu(jnp.dot(comb, w1_ref[...],
                              preferred_element_type=jnp.float32) + b1_ref[...])
    out_ref[...] = jnp.dot(hid, w2_ref[...],
                           preferred_element_type=jnp.float32) + b2_ref[...]


def kernel(x_text_indices, node_selectors, edge_index, edge_type,
           edge_positions, emb, pos_table, Wp, bp, We, be, W_ih, W_hh,
           b_ih, b_hh, W1, b1, W2, b2):
    n_nodes = x_text_indices.shape[0]
    nb = n_nodes // BLK
    src = edge_index[0].astype(jnp.int32)
    dst = edge_index[1].astype(jnp.int32)
    idx3 = dst * NTYPES + edge_type.astype(jnp.int32)
    pos = edge_positions.astype(jnp.int32)

    ptab = jnp.pad(pos_table, ((0, SROW - pos_table.shape[0]), (0, 0)))
    gate_tab = pl.pallas_call(
        _gate_tab_body,
        out_shape=jax.ShapeDtypeStruct((SROW, SROW), jnp.float32),
    )(ptab, Wp, bp.reshape(1, HID))

    h0 = jnp.concatenate(
        [jnp.take(emb, x_text_indices, axis=0), node_selectors,
         jnp.ones((n_nodes, 1), jnp.float32),
         jnp.zeros((n_nodes, SROW - HID - 1), jnp.float32)], axis=1)

    zeros_in = jnp.zeros((ZROWS, SROW), jnp.float32)
    edge_fn = _make_edge_fn()

    node_fn = pl.pallas_call(
        _node_body,
        grid=(nb,),
        in_specs=[
            pl.BlockSpec((BLK, NTYPES, SROW), lambda i: (i, 0, 0)),
            pl.BlockSpec((BLK, SROW), lambda i: (i, 0)),
            pl.BlockSpec((NTYPES, HID, HID), lambda i: (0, 0, 0)),
            pl.BlockSpec((NTYPES, HID), lambda i: (0, 0)),
            pl.BlockSpec((HID, 3 * HID), lambda i: (0, 0)),
            pl.BlockSpec((HID, 3 * HID), lambda i: (0, 0)),
            pl.BlockSpec((1, 3 * HID), lambda i: (0, 0)),
            pl.BlockSpec((1, 3 * HID), lambda i: (0, 0)),
        ],
        out_specs=pl.BlockSpec((BLK, SROW), lambda i: (i, 0)),
        out_shape=jax.ShapeDtypeStruct((n_nodes, SROW), jnp.float32),
    )

    h = h0
    wih_t = W_ih.T
    whh_t = W_hh.T
    for _ in range(ITERS):
        s_flat = edge_fn(idx3, src, pos, h, gate_tab, zeros_in)
        s = s_flat.reshape(S_ROWS // NTYPES, NTYPES, SROW)
        h = node_fn(s, h, We, be, wih_t, whh_t,
                    b_ih.reshape(1, -1), b_hh.reshape(1, -1))

    logits = pl.pallas_call(
        _mlp_body,
        grid=(nb,),
        in_specs=[
            pl.BlockSpec((BLK, SROW), lambda i: (i, 0)),
            pl.BlockSpec((BLK, SROW), lambda i: (i, 0)),
            pl.BlockSpec((2 * HID, HID), lambda i: (0, 0)),
            pl.BlockSpec((1, HID), lambda i: (0, 0)),
            pl.BlockSpec((HID, 1), lambda i: (0, 0)),
            pl.BlockSpec((1, 1), lambda i: (0, 0)),
        ],
        out_specs=pl.BlockSpec((BLK, 1), lambda i: (i, 0)),
        out_shape=jax.ShapeDtypeStruct((n_nodes, 1), jnp.float32),
    )(h, h0, W1, b1.reshape(1, -1), W2, b2.reshape(1, 1))

    return logits
